# baseline (speedup 1.0000x reference)
import functools

import jax
import jax.numpy as jnp
from jax.experimental import pallas as pl
from jax.experimental.pallas import tpu as pltpu

EPS = 1e-08


def _main_body(attn_ref, feat_ref, hc_ref, wn_ref, bn_ref, wnb_ref, bnb_ref,
               alpha_ref, out_ref, *, bm):
    i = pl.program_id(0)
    nbf = jnp.dot(attn_ref[...], feat_ref[...],
                  preferred_element_type=jnp.float32)
    nb = jnp.dot(nbf, wnb_ref[...], preferred_element_type=jnp.float32)
    nb = nb + bnb_ref[...]
    fi = feat_ref[pl.ds(i * bm, bm), :]
    node = jnp.dot(fi, wn_ref[...], preferred_element_type=jnp.float32)
    node = node + bn_ref[...]
    gate = jax.nn.sigmoid(alpha_ref[0, 0])
    out_ref[...] = node + (gate * hc_ref[...]) * nb


def _norm_body(x_ref, g_ref, b_ref, o_ref, *, n):
    x = x_ref[...]
    mean = jnp.mean(x, axis=0, keepdims=True)
    d = x - mean
    var = jnp.sum(d * d, axis=0, keepdims=True) / (n - 1)
    std = jnp.sqrt(var)
    y = g_ref[...] * (d / (std + EPS)) + b_ref[...]
    o_ref[...] = jnp.maximum(y, 0.0)


def kernel(feat, edge_index, attn_matrix, has_cn, W_node, b_node, W_nb, b_nb,
           alpha, gamma, beta):
    n, d_in = feat.shape
    d_out = W_node.shape[1]
    bm = 200
    assert n % bm == 0
    const = lambda i: (0, 0)
    out_pre = pl.pallas_call(
        functools.partial(_main_body, bm=bm),
        grid=(n // bm,),
        in_specs=[
            pl.BlockSpec((bm, n), lambda i: (i, 0)),
            pl.BlockSpec((n, d_in), const),
            pl.BlockSpec((bm, 1), lambda i: (i, 0)),
            pl.BlockSpec((d_in, d_out), const),
            pl.BlockSpec((1, d_out), const),
            pl.BlockSpec((d_in, d_out), const),
            pl.BlockSpec((1, d_out), const),
            pl.BlockSpec((1, 1), const),
        ],
        out_specs=pl.BlockSpec((bm, d_out), lambda i: (i, 0)),
        out_shape=jax.ShapeDtypeStruct((n, d_out), jnp.float32),
        compiler_params=pltpu.CompilerParams(
            dimension_semantics=("parallel",)),
    )(attn_matrix, feat, has_cn, W_node, b_node.reshape(1, d_out),
      W_nb, b_nb.reshape(1, d_out), alpha.reshape(1, 1))

    out = pl.pallas_call(
        functools.partial(_norm_body, n=n),
        out_shape=jax.ShapeDtypeStruct((n, d_out), jnp.float32),
    )(out_pre, gamma.reshape(1, d_out), beta.reshape(1, d_out))
    return out
